# baseline (speedup 1.0000x reference)
import dataclasses
import functools
import jax
import jax.numpy as jnp
from jax import lax
from jax.experimental import pallas as pl
from jax.experimental.pallas import tpu as pltpu
from jax.experimental.pallas import tpu_sc as plsc

B = 16384
D = 32
NC = 2
NS = 16
L = 16
NW = NC * NS
BPW = B // NW
CHUNK = 128
NCHUNK = BPW // CHUNK


def _sc_kernel(users_hbm, items_hbm, utab_hbm, itab_hbm, out_hbm,
               uidx_v, iidx_v, urows_v, irows_v, out_v, sem):
    wid = lax.axis_index("s") * NC + lax.axis_index("c")
    base = wid * BPW
    pltpu.sync_copy(users_hbm.at[pl.ds(base, BPW)], uidx_v)
    pltpu.sync_copy(items_hbm.at[pl.ds(base, BPW)], iidx_v)

    for c in range(NCHUNK):
        cu = pltpu.async_copy(
            utab_hbm.at[uidx_v.at[pl.ds(c * CHUNK, CHUNK)]], urows_v, sem)
        ci = pltpu.async_copy(
            itab_hbm.at[iidx_v.at[pl.ds(c * CHUNK, CHUNK)]], irows_v, sem)
        cu.wait()
        ci.wait()

        @pl.loop(0, CHUNK // L)
        def _(g):
            rowi = lax.iota(jnp.int32, L) + g * L
            acc = jnp.zeros((L,), jnp.float32)
            for d in range(D):
                cold = jnp.full((L,), d, jnp.int32)
                uvals = plsc.load_gather(urows_v, [rowi, cold])
                ivals = plsc.load_gather(irows_v, [rowi, cold])
                acc = acc + uvals * ivals
            sig = 1.0 / (1.0 + jnp.exp(-acc))
            out_v[pl.ds(c * CHUNK + g * L, L)] = sig

    pltpu.sync_copy(out_v, out_hbm.at[pl.ds(base, BPW)])


@jax.jit
def _run(users, items, user_table, item_table):
    mesh = plsc.VectorSubcoreMesh(core_axis_name="c", subcore_axis_name="s")
    cp = pltpu.CompilerParams(
        needs_layout_passes=False, use_tc_tiling_on_sc=False)
    k = pl.kernel(
        _sc_kernel,
        out_type=jax.ShapeDtypeStruct((B,), jnp.float32),
        mesh=mesh,
        scratch_types=[
            pltpu.VMEM((BPW,), jnp.int32),
            pltpu.VMEM((BPW,), jnp.int32),
            pltpu.VMEM((CHUNK, D), jnp.float32),
            pltpu.VMEM((CHUNK, D), jnp.float32),
            pltpu.VMEM((BPW,), jnp.float32),
            pltpu.SemaphoreType.DMA,
        ],
        compiler_params=cp,
    )
    return k(users, items, user_table, item_table)


def kernel(users, items, user_table, item_table):
    return _run(users, items, user_table, item_table)
